# baseline (speedup 1.0000x reference)
import functools
import numpy as np
import jax
import jax.numpy as jnp
from jax import lax
from jax.experimental import pallas as pl
from jax.experimental.pallas import tpu as pltpu
from jax.experimental.pallas import tpu_sc as plsc

_N = 10000
_E = 320000
_NPAD = 10240
_NW = 32
_EPW = _E // _NW
_CH = 128
_NFULL = _EPW // _CH
_TAILOFF = _NFULL * _CH
_TAIL = _EPW - _TAILOFF
_RPS = _NPAD // 16
_EBLK = 2000
_HI = lax.Precision.HIGHEST

_f32 = jnp.float32
_i32 = jnp.int32


def _node_prelude_body(x_ref, w0_ref, b0_ref, nw_ref, nb_ref, nms_ref,
                       w1_ref, b1_ref, xn_ref, h_ref):
    x1 = jnp.dot(x_ref[...], w0_ref[...], precision=_HI) + b0_ref[...]
    mean = jnp.mean(x1, axis=0, keepdims=True)
    o = x1 - mean * nms_ref[...]
    var = jnp.mean(o * o, axis=0, keepdims=True)
    xn = nw_ref[...] * o / jnp.sqrt(var + 1e-5) + nb_ref[...]
    h = jnp.maximum(jnp.dot(xn, w1_ref[...], precision=_HI) + b1_ref[...], 0.0)
    zrows = jnp.zeros((_NPAD - _N, 128), _f32)
    zrows16 = jnp.zeros((_NPAD - _N, 16), _f32)
    xn_ref[...] = jnp.concatenate([xn, zrows], axis=0)
    h_ref[...] = jnp.concatenate(
        [jnp.concatenate([xn, h], axis=1),
         jnp.concatenate([zrows, zrows16], axis=1)], axis=0)


def _edge_stats_body(ea_ref, w_ref, b_ref, out_ref, st_ref):
    i = pl.program_id(0)
    ea = jnp.dot(ea_ref[...], w_ref[...], precision=_HI) + b_ref[...]
    out_ref[...] = ea

    @pl.when(i == 0)
    def _():
        st_ref[...] = jnp.zeros_like(st_ref)

    s = jnp.sum(ea, axis=0, keepdims=True)
    s2 = jnp.sum(ea * ea, axis=0, keepdims=True)
    st_ref[...] += jnp.concatenate(
        [s, s2, jnp.zeros((6, 16), _f32)], axis=0)


def _edge_proj_body(ea_ref, st_ref, n2w_ref, n2b_ref, n2ms_ref,
                    eaw_ref, eab_ref, e1w_ref, e1b_ref, e2w_ref, e2b_ref,
                    we2_ref, we3_ref, we4_ref, we5_ref,
                    o2_ref, o3_ref, o4_ref, o5_ref):
    st = st_ref[...]
    m = st[0:1, :] / _E
    msq = st[1:2, :] / _E
    ms = n2ms_ref[...]
    var = msq - 2.0 * ms * m * m + ms * ms * m * m
    o = ea_ref[...] - m * ms
    ean = n2w_ref[...] * o / jnp.sqrt(var + 1e-5) + n2b_ref[...]
    ea_b = jnp.dot(ean, eaw_ref[...], precision=_HI) + eab_ref[...]
    h10 = jnp.maximum(jnp.dot(ean, e1w_ref[...], precision=_HI) + e1b_ref[...], 0.0)
    ea10 = jnp.dot(h10, e2w_ref[...], precision=_HI) + e2b_ref[...]
    col = lax.broadcasted_iota(_i32, (_EBLK, 16), 1)
    for dst_ref, src, w_ref in ((o2_ref, ea10, we2_ref), (o3_ref, ea10, we3_ref),
                                (o4_ref, ea_b, we4_ref), (o5_ref, ea_b, we5_ref)):
        ep = jnp.dot(src, w_ref[...], precision=_HI)
        dst_ref[...] = jnp.where(col == 10, 1.0, ep)


def _proj_core(h, w_ref, b_ref, q_ref, kv_ref, s_ref):
    o = jnp.dot(h, w_ref[...], precision=_HI) + b_ref[...]
    q_ref[...] = o[:, 0:16]
    kv_ref[...] = o[:, 16:48]
    s_ref[...] = o[:, 48:64]


def _proj_body(h_ref, w_ref, b_ref, q_ref, kv_ref, s_ref):
    _proj_core(h_ref[...], w_ref, b_ref, q_ref, kv_ref, s_ref)


def _combine16(acc_ref, s_ref):
    num = acc_ref[0] + acc_ref[1]
    den = num[:, 10:11]
    return num / (den + 1e-16) + s_ref[...]


def _combine_proj_body(acc_ref, s_ref, xn_ref, w_ref, b_ref,
                       q_ref, kv_ref, s2_ref):
    o16 = jnp.maximum(_combine16(acc_ref, s_ref), 0.0)
    h = jnp.concatenate([xn_ref[...], o16], axis=1)
    _proj_core(h, w_ref, b_ref, q_ref, kv_ref, s2_ref)


def _combine_final_body(acc_ref, s_ref, lw_ref, lb_ref, out_ref):
    o16 = _combine16(acc_ref, s_ref)
    out_ref[...] = jnp.dot(o16, lw_ref[...], precision=_HI) + lb_ref[...]


def _softmax_body(g_ref, out_ref):
    g = g_ref[...]
    m = jnp.max(g, axis=1, keepdims=True)
    e = jnp.exp(g - m)
    out_ref[...] = e / jnp.sum(e, axis=1, keepdims=True)


@functools.cache
def _sc_mesh():
    return plsc.VectorSubcoreMesh(core_axis_name="c", subcore_axis_name="s")


def _edge_pass_body(src_hbm, dst_hbm, kv_hbm, q_hbm, ep_hbm, out_hbm,
                    srcv0, srcv1, dstv0, dstv1, dsts0, dsts1,
                    kvv0, kvv1, qv0, qv1,
                    epv0, epv1, outb0, outb1,
                    srct, dstt, kvt, qt, ept, outt,
                    zbuf, semi0, semi1, semk0, semk1, semq0, semq1,
                    seme0, seme1, sems0, sems1, semt, kv_s, q_s, accs):
    c = lax.axis_index("c")
    s = lax.axis_index("s")
    wid = s * 2 + c
    base = wid * _EPW
    lanes = lax.iota(_i32, 16)
    srcv = (srcv0, srcv1)
    dstv = (dstv0, dstv1)
    dsts = (dsts0, dsts1)
    kvv = (kvv0, kvv1)
    qv = (qv0, qv1)
    epv = (epv0, epv1)
    outb = (outb0, outb1)
    semi = (semi0, semi1)
    semk = (semk0, semk1)
    semq = (semq0, semq1)
    seme = (seme0, seme1)
    sems = (sems0, sems1)

    def _zrow(i, carry):
        zbuf[i, :] = jnp.zeros((16,), _f32)
        return carry

    lax.fori_loop(0, _RPS, _zrow, 0)
    pltpu.sync_copy(zbuf, accs.at[pl.ds(s * _RPS, _RPS)])
    pltpu.sync_copy(kv_hbm.at[pl.ds(s * _RPS, _RPS)], kv_s.at[pl.ds(s * _RPS, _RPS)])
    pltpu.sync_copy(q_hbm.at[pl.ds(s * _RPS, _RPS)], q_s.at[pl.ds(s * _RPS, _RPS)])
    for b in (0, 1):
        def _z16(i, carry, _b=b):
            outb[_b][i, :] = jnp.zeros((16,), _f32)
            return carry
        lax.fori_loop(0, _CH, _z16, 0)
    plsc.subcore_barrier()

    def issue_idx(j, b):
        off = base + j * _CH
        pltpu.async_copy(src_hbm.at[pl.ds(off, _CH)], srcv[b], semi[b])
        pltpu.async_copy(dst_hbm.at[pl.ds(off, _CH)], dstv[b], semi[b])
        pltpu.async_copy(ep_hbm.at[pl.ds(off, _CH)], epv[b], seme[b])

    def wait_idx(b):
        pltpu.make_async_copy(src_hbm.at[pl.ds(base, _CH)], srcv[b], semi[b]).wait()
        pltpu.make_async_copy(dst_hbm.at[pl.ds(base, _CH)], dstv[b], semi[b]).wait()

    def issue_gather(b):
        pltpu.async_copy(kv_s.at[srcv[b]], kvv[b], semk[b])
        pltpu.async_copy(q_s.at[dstv[b]], qv[b], semq[b])

    def wait_gather(b):
        pltpu.make_async_copy(kv_s.at[srcv[b]], kvv[b], semk[b]).wait()
        pltpu.make_async_copy(q_s.at[dstv[b]], qv[b], semq[b]).wait()
        pltpu.make_async_copy(ep_hbm.at[pl.ds(base, _CH)], epv[b], seme[b]).wait()

    def wait_scatter(b):
        pltpu.make_async_copy(outb[b], accs.at[dsts[b]], sems[b]).wait()

    def issue_scatter(b):
        for g in range(_CH // 16):
            dsts[b][pl.ds(g * 16, 16)] = dstv[b][pl.ds(g * 16, 16)]
        pltpu.async_copy(outb[b], accs.at[dsts[b]], sems[b], add=True)

    def compute_groups(q_r, kv_r, ep_r, o_r, ngroups):
        for g in range(ngroups):
            eidx = lanes + (g * 16)
            alpha = jnp.zeros((16,), _f32)
            ve = []
            for j in range(10):
                jj = jnp.full((16,), j, _i32)
                qj = plsc.load_gather(q_r, [eidx, jj])
                kj = plsc.load_gather(kv_r, [eidx, jj])
                ej = plsc.load_gather(ep_r, [eidx, jj])
                vj = plsc.load_gather(kv_r, [eidx, jj + 16])
                alpha = alpha + qj * (kj + ej)
                ve.append(vj + ej)
            ex = jnp.exp(alpha)
            for j in range(10):
                jj = jnp.full((16,), j, _i32)
                plsc.store_scatter(o_r, [eidx, jj], ex * ve[j])
            plsc.store_scatter(o_r, [eidx, jnp.full((16,), 10, _i32)], ex)

    issue_idx(0, 0)
    issue_idx(1, 1)
    wait_idx(0)
    issue_gather(0)

    def _pair(j2, carry):
        for b in (0, 1):
            j = j2 * 2 + b
            nb = 1 - b
            wait_idx(nb)
            issue_gather(nb)
            wait_gather(b)

            @pl.when(j >= 2)
            def _():
                wait_scatter(b)

            compute_groups(qv[b], kvv[b], epv[b], outb[b], 8)
            issue_scatter(b)

            @pl.when(j + 2 < _NFULL)
            def _():
                issue_idx(j + 2, b)
        return carry

    lax.fori_loop(0, _NFULL // 2 - 1, _pair, 0)
    for j in (_NFULL - 2, _NFULL - 1):
        b = j % 2
        nb = 1 - b
        if j == _NFULL - 2:
            wait_idx(nb)
            issue_gather(nb)
        wait_gather(b)
        wait_scatter(b)
        compute_groups(qv[b], kvv[b], epv[b], outb[b], 8)
        issue_scatter(b)

    toff = base + _TAILOFF
    pltpu.sync_copy(src_hbm.at[pl.ds(toff, _TAIL)], srct)
    pltpu.sync_copy(dst_hbm.at[pl.ds(toff, _TAIL)], dstt)
    g1 = pltpu.async_copy(kv_s.at[srct], kvt, semt)
    pltpu.sync_copy(ep_hbm.at[pl.ds(toff, _TAIL)], ept)
    g1.wait()
    pltpu.async_copy(q_s.at[dstt], qt, semt).wait()
    def _zt(i, carry):
        outt[i, :] = jnp.zeros((16,), _f32)
        return carry
    lax.fori_loop(0, _TAIL, _zt, 0)
    compute_groups(qt, kvt, ept, outt, 1)
    wait_scatter(0)
    wait_scatter(1)
    pltpu.sync_copy(outt, accs.at[dstt], add=True)

    plsc.subcore_barrier()
    pltpu.sync_copy(accs.at[pl.ds(s * _RPS, _RPS)], zbuf)
    pltpu.sync_copy(zbuf, out_hbm.at[c, pl.ds(s * _RPS, _RPS)])


@functools.cache
def _edge_pass():
    return pl.kernel(
    _edge_pass_body,
    out_type=jax.ShapeDtypeStruct((2, _NPAD, 16), _f32),
    mesh=_sc_mesh(),
    scratch_types=(
        [pltpu.VMEM((_CH,), _i32)] * 6
        + [pltpu.VMEM((_CH, 32), _f32)] * 2
        + [pltpu.VMEM((_CH, 16), _f32)] * 2
        + [pltpu.VMEM((_CH, 16), _f32)] * 2
        + [pltpu.VMEM((_CH, 16), _f32)] * 2
        + [pltpu.VMEM((_TAIL,), _i32)] * 2
        + [pltpu.VMEM((_TAIL, 32), _f32)]
        + [pltpu.VMEM((_TAIL, 16), _f32)] * 3
        + [pltpu.VMEM((_RPS, 16), _f32)]
        + [pltpu.SemaphoreType.DMA] * 11
        + [pltpu.VMEM_SHARED((_NPAD, 32), _f32)]
        + [pltpu.VMEM_SHARED((_NPAD, 16), _f32)] * 2
    ),
    compiler_params=pltpu.CompilerParams(
        needs_layout_passes=False, use_tc_tiling_on_sc=False),
    )


def _pick_body(tab_hbm, idx_hbm, out_hbm, idxv, rowsv, sem):
    c = lax.axis_index("c")
    s = lax.axis_index("s")
    wid = s * 2 + c
    base = wid * 32
    pltpu.sync_copy(idx_hbm.at[pl.ds(base, 32)], idxv)
    pltpu.async_copy(tab_hbm.at[idxv], rowsv, sem).wait()
    pltpu.sync_copy(rowsv, out_hbm.at[pl.ds(base, 32)])


@functools.cache
def _pick():
    return pl.kernel(
        _pick_body,
        out_type=jax.ShapeDtypeStruct((1024, 16), _f32),
        mesh=_sc_mesh(),
        scratch_types=[
            pltpu.VMEM((32,), _i32),
            pltpu.VMEM((32, 16), _f32),
            pltpu.SemaphoreType.DMA,
        ],
        compiler_params=pltpu.CompilerParams(
            needs_layout_passes=False, use_tc_tiling_on_sc=False),
    )


def _pad_cols(w, cols):
    return jnp.concatenate([w, jnp.zeros((w.shape[0], cols - w.shape[1]), _f32)], axis=1)


def _qkvs_weights(tc):
    rs = np.float32(1.0 / np.sqrt(10.0))
    z6 = jnp.zeros((138, 6), _f32)
    w = jnp.concatenate(
        [tc["Wq"] * rs, z6, tc["Wk"], z6, tc["Wv"], z6, tc["Ws"], z6], axis=1)
    w = jnp.concatenate([w, jnp.zeros((6, 64), _f32)], axis=0)
    z6b = jnp.zeros((6,), _f32)
    b = jnp.concatenate(
        [tc["bq"] * rs, z6b, tc["bk"], z6b, tc["bv"], z6b, tc["bs"], z6b])
    return w, b.reshape(1, 64)


def kernel(x, z, edge_index, z1edge_index, z2edge_index, z3edge_index,
           z4edge_index, z5edge_index, edge_attr, pickable, params):
    p = params
    f32 = _f32

    l1w = _pad_cols(p["l1_W"], 16)
    l1b = _pad_cols(p["l1_b"].reshape(1, 10), 16)
    eaw = _pad_cols(p["eA_W"], 16)
    eab = _pad_cols(p["eA_b"].reshape(1, 10), 16)
    e1w = _pad_cols(p["e1_W"], 16)
    e1b = _pad_cols(p["e1_b"].reshape(1, 10), 16)
    e2w = jnp.zeros((16, 16), f32).at[:10, :10].set(p["e2_W"])
    e2b = _pad_cols(p["e2_b"].reshape(1, 10), 16)
    wes = {l: jnp.zeros((16, 16), f32).at[:10, :10].set(p[f"tc{l}"]["We"])
           for l in (2, 3, 4, 5)}
    qkvs = {l: _qkvs_weights(p[f"tc{l}"]) for l in (2, 3, 4, 5)}
    linw = jnp.zeros((16, 16), f32).at[:10, :10].set(p["lin_W"])
    linb = jnp.full((1, 16), -1e30, f32).at[0, :10].set(p["lin_b"])

    xn, h144 = pl.pallas_call(
        _node_prelude_body,
        out_shape=[jax.ShapeDtypeStruct((_NPAD, 128), f32),
                   jax.ShapeDtypeStruct((_NPAD, 144), f32)],
    )(x, p["l0_W"], p["l0_b"].reshape(1, 128), p["n_w"].reshape(1, 128),
      p["n_b"].reshape(1, 128), p["n_ms"].reshape(1, 128), l1w, l1b)

    ngrid = _E // _EBLK
    blk = pl.BlockSpec((_EBLK, 16), lambda i: (i, 0))
    wspec = pl.BlockSpec((16, 16), lambda i: (0, 0))
    bspec = pl.BlockSpec((1, 16), lambda i: (0, 0))
    ea, stats = pl.pallas_call(
        _edge_stats_body,
        grid=(ngrid,),
        in_specs=[blk, wspec, bspec],
        out_specs=[blk, pl.BlockSpec((8, 16), lambda i: (0, 0))],
        out_shape=[jax.ShapeDtypeStruct((_E, 16), f32),
                   jax.ShapeDtypeStruct((8, 16), f32)],
    )(edge_attr, _pad_cols(p["l0e_W"], 16)[:16, :],
      _pad_cols(p["l0e_b"].reshape(1, 16), 16))

    eps = pl.pallas_call(
        _edge_proj_body,
        grid=(ngrid,),
        in_specs=[blk, pl.BlockSpec((8, 16), lambda i: (0, 0)),
                  bspec, bspec, bspec,
                  wspec, bspec, wspec, bspec, wspec, bspec,
                  wspec, wspec, wspec, wspec],
        out_specs=[blk, blk, blk, blk],
        out_shape=[jax.ShapeDtypeStruct((_E, 16), f32)] * 4,
    )(ea, stats,
      _pad_cols(p["n2_w"].reshape(1, 16), 16),
      _pad_cols(p["n2_b"].reshape(1, 16), 16),
      _pad_cols(p["n2_ms"].reshape(1, 16), 16),
      eaw, eab, e1w, e1b, e2w, e2b,
      wes[2], wes[3], wes[4], wes[5])

    src = edge_index[0]
    dst = edge_index[1]

    nblk = 2048
    rgrid = (_NPAD // nblk,)
    b144 = pl.BlockSpec((nblk, 144), lambda i: (i, 0))
    b128 = pl.BlockSpec((nblk, 128), lambda i: (i, 0))
    b64w = pl.BlockSpec((144, 64), lambda i: (0, 0))
    b64b = pl.BlockSpec((1, 64), lambda i: (0, 0))
    b16 = pl.BlockSpec((nblk, 16), lambda i: (i, 0))
    b32 = pl.BlockSpec((nblk, 32), lambda i: (i, 0))
    bacc = pl.BlockSpec((2, nblk, 16), lambda i: (0, i, 0))
    w16 = pl.BlockSpec((16, 16), lambda i: (0, 0))
    w1x16 = pl.BlockSpec((1, 16), lambda i: (0, 0))
    proj_out_shape = [jax.ShapeDtypeStruct((_NPAD, 16), f32),
                      jax.ShapeDtypeStruct((_NPAD, 32), f32),
                      jax.ShapeDtypeStruct((_NPAD, 16), f32)]
    proj_out_specs = [b16, b32, b16]

    q_t, kv_t, s_t = pl.pallas_call(
        _proj_body,
        grid=rgrid,
        in_specs=[b144, b64w, b64b],
        out_specs=proj_out_specs,
        out_shape=proj_out_shape,
    )(h144, qkvs[2][0], qkvs[2][1])

    for l in (2, 3, 4):
        acc = _edge_pass()(src, dst, kv_t, q_t, eps[l - 2])
        q_t, kv_t, s_t = pl.pallas_call(
            _combine_proj_body,
            grid=rgrid,
            in_specs=[bacc, b16, b128, b64w, b64b],
            out_specs=proj_out_specs,
            out_shape=proj_out_shape,
        )(acc, s_t, xn, qkvs[l + 1][0], qkvs[l + 1][1])

    acc = _edge_pass()(src, dst, kv_t, q_t, eps[3])
    logits = pl.pallas_call(
        _combine_final_body,
        grid=rgrid,
        in_specs=[bacc, b16, w16, w1x16],
        out_specs=b16,
        out_shape=jax.ShapeDtypeStruct((_NPAD, 16), f32),
    )(acc, s_t, linw, linb)

    pick_pad = jnp.concatenate([pickable, jnp.zeros((24,), _i32)])
    picked = _pick()(logits, pick_pad)
    sm = pl.pallas_call(
        _softmax_body,
        out_shape=jax.ShapeDtypeStruct((1024, 16), f32),
    )(picked)
    return sm[:1000, :10]


# speedup vs baseline: 1.0213x; 1.0213x over previous
import functools
import numpy as np
import jax
import jax.numpy as jnp
from jax import lax
from jax.experimental import pallas as pl
from jax.experimental.pallas import tpu as pltpu
from jax.experimental.pallas import tpu_sc as plsc

_N = 10000
_E = 320000
_NPAD = 10240
_NW = 32
_EPW = _E // _NW
_CH = 128
_NFULL = _EPW // _CH
_TAILOFF = _NFULL * _CH
_TAIL = _EPW - _TAILOFF
_RPS = _NPAD // 16
_EBLK = 2000
_HI = lax.Precision.HIGHEST

_f32 = jnp.float32
_i32 = jnp.int32


def _node_prelude_body(x_ref, w0_ref, b0_ref, nw_ref, nb_ref, nms_ref,
                       w1_ref, b1_ref, xn_ref, h_ref):
    x1 = jnp.dot(x_ref[...], w0_ref[...], precision=_HI) + b0_ref[...]
    mean = jnp.mean(x1, axis=0, keepdims=True)
    o = x1 - mean * nms_ref[...]
    var = jnp.mean(o * o, axis=0, keepdims=True)
    xn = nw_ref[...] * o / jnp.sqrt(var + 1e-5) + nb_ref[...]
    h = jnp.maximum(jnp.dot(xn, w1_ref[...], precision=_HI) + b1_ref[...], 0.0)
    zrows = jnp.zeros((_NPAD - _N, 128), _f32)
    zrows16 = jnp.zeros((_NPAD - _N, 16), _f32)
    xn_ref[...] = jnp.concatenate([xn, zrows], axis=0)
    h_ref[...] = jnp.concatenate(
        [jnp.concatenate([xn, h], axis=1),
         jnp.concatenate([zrows, zrows16], axis=1)], axis=0)


def _edge_stats_body(ea_ref, w_ref, b_ref, out_ref, st_ref):
    i = pl.program_id(0)
    ea = jnp.dot(ea_ref[...], w_ref[...], precision=_HI) + b_ref[...]
    out_ref[...] = ea

    @pl.when(i == 0)
    def _():
        st_ref[...] = jnp.zeros_like(st_ref)

    s = jnp.sum(ea, axis=0, keepdims=True)
    s2 = jnp.sum(ea * ea, axis=0, keepdims=True)
    st_ref[...] += jnp.concatenate(
        [s, s2, jnp.zeros((6, 16), _f32)], axis=0)


def _edge_proj_body(ea_ref, st_ref, n2w_ref, n2b_ref, n2ms_ref,
                    eaw_ref, eab_ref, e1w_ref, e1b_ref, e2w_ref, e2b_ref,
                    we2_ref, we3_ref, we4_ref, we5_ref,
                    o2_ref, o3_ref, o4_ref, o5_ref):
    st = st_ref[...]
    m = st[0:1, :] / _E
    msq = st[1:2, :] / _E
    ms = n2ms_ref[...]
    var = msq - 2.0 * ms * m * m + ms * ms * m * m
    o = ea_ref[...] - m * ms
    ean = n2w_ref[...] * o / jnp.sqrt(var + 1e-5) + n2b_ref[...]
    ea_b = jnp.dot(ean, eaw_ref[...], precision=_HI) + eab_ref[...]
    h10 = jnp.maximum(jnp.dot(ean, e1w_ref[...], precision=_HI) + e1b_ref[...], 0.0)
    ea10 = jnp.dot(h10, e2w_ref[...], precision=_HI) + e2b_ref[...]
    col = lax.broadcasted_iota(_i32, (_EBLK, 16), 1)
    for dst_ref, src, w_ref in ((o2_ref, ea10, we2_ref), (o3_ref, ea10, we3_ref),
                                (o4_ref, ea_b, we4_ref), (o5_ref, ea_b, we5_ref)):
        ep = jnp.dot(src, w_ref[...], precision=_HI)
        dst_ref[...] = jnp.where(col == 10, 1.0, ep)


def _proj_core(h, w_ref, b_ref, q_ref, kv_ref, s_ref):
    o = jnp.dot(h, w_ref[...], precision=_HI) + b_ref[...]
    q_ref[...] = o[:, 0:16]
    kv_ref[...] = o[:, 16:48]
    s_ref[...] = o[:, 48:64]


def _proj_body(h_ref, w_ref, b_ref, q_ref, kv_ref, s_ref):
    _proj_core(h_ref[...], w_ref, b_ref, q_ref, kv_ref, s_ref)


def _combine16(acc_ref, s_ref):
    num = acc_ref[0] + acc_ref[1]
    den = num[:, 10:11]
    return num / (den + 1e-16) + s_ref[...]


def _combine_proj_body(acc_ref, s_ref, xn_ref, w_ref, b_ref,
                       q_ref, kv_ref, s2_ref):
    o16 = jnp.maximum(_combine16(acc_ref, s_ref), 0.0)
    h = jnp.concatenate([xn_ref[...], o16], axis=1)
    _proj_core(h, w_ref, b_ref, q_ref, kv_ref, s2_ref)


def _combine_final_body(acc_ref, s_ref, lw_ref, lb_ref, out_ref):
    o16 = _combine16(acc_ref, s_ref)
    out_ref[...] = jnp.dot(o16, lw_ref[...], precision=_HI) + lb_ref[...]


def _softmax_body(g_ref, out_ref):
    g = g_ref[...]
    m = jnp.max(g, axis=1, keepdims=True)
    e = jnp.exp(g - m)
    out_ref[...] = e / jnp.sum(e, axis=1, keepdims=True)


@functools.cache
def _sc_mesh():
    return plsc.VectorSubcoreMesh(core_axis_name="c", subcore_axis_name="s")


def _edge_pass_body(src_hbm, dst_hbm, kv_hbm, q_hbm, ep_hbm, out_hbm,
                    srcv0, srcv1, srcv2, dstv0, dstv1, dstv2,
                    dsts0, dsts1, dsts2,
                    kvv0, kvv1, kvv2, qv0, qv1, qv2,
                    epv0, epv1, epv2, outb0, outb1, outb2,
                    srct, dstt, kvt, qt, ept, outt,
                    zbuf, semi0, semi1, semi2, semk0, semk1, semk2,
                    semq0, semq1, semq2, seme0, seme1, seme2,
                    sems0, sems1, sems2, semt, kv_s, q_s, accs):
    c = lax.axis_index("c")
    s = lax.axis_index("s")
    wid = s * 2 + c
    base = wid * _EPW
    lanes = lax.iota(_i32, 16)
    srcv = (srcv0, srcv1, srcv2)
    dstv = (dstv0, dstv1, dstv2)
    dsts = (dsts0, dsts1, dsts2)
    kvv = (kvv0, kvv1, kvv2)
    qv = (qv0, qv1, qv2)
    epv = (epv0, epv1, epv2)
    outb = (outb0, outb1, outb2)
    semi = (semi0, semi1, semi2)
    semk = (semk0, semk1, semk2)
    semq = (semq0, semq1, semq2)
    seme = (seme0, seme1, seme2)
    sems = (sems0, sems1, sems2)

    def _zrow(i, carry):
        zbuf[i, :] = jnp.zeros((16,), _f32)
        return carry

    lax.fori_loop(0, _RPS, _zrow, 0)
    pltpu.sync_copy(zbuf, accs.at[pl.ds(s * _RPS, _RPS)])
    pltpu.sync_copy(kv_hbm.at[pl.ds(s * _RPS, _RPS)], kv_s.at[pl.ds(s * _RPS, _RPS)])
    pltpu.sync_copy(q_hbm.at[pl.ds(s * _RPS, _RPS)], q_s.at[pl.ds(s * _RPS, _RPS)])
    for b in (0, 1, 2):
        def _z16(i, carry, _b=b):
            outb[_b][i, :] = jnp.zeros((16,), _f32)
            return carry
        lax.fori_loop(0, _CH, _z16, 0)
    plsc.subcore_barrier()

    def issue_idx(j, b):
        off = base + j * _CH
        pltpu.async_copy(src_hbm.at[pl.ds(off, _CH)], srcv[b], semi[b])
        pltpu.async_copy(dst_hbm.at[pl.ds(off, _CH)], dstv[b], semi[b])
        pltpu.async_copy(ep_hbm.at[pl.ds(off, _CH)], epv[b], seme[b])

    def wait_idx(b):
        pltpu.make_async_copy(src_hbm.at[pl.ds(base, _CH)], srcv[b], semi[b]).wait()
        pltpu.make_async_copy(dst_hbm.at[pl.ds(base, _CH)], dstv[b], semi[b]).wait()

    def issue_gather(b):
        pltpu.async_copy(kv_s.at[srcv[b]], kvv[b], semk[b])
        pltpu.async_copy(q_s.at[dstv[b]], qv[b], semq[b])

    def wait_gather(b):
        pltpu.make_async_copy(kv_s.at[srcv[b]], kvv[b], semk[b]).wait()
        pltpu.make_async_copy(q_s.at[dstv[b]], qv[b], semq[b]).wait()
        pltpu.make_async_copy(ep_hbm.at[pl.ds(base, _CH)], epv[b], seme[b]).wait()

    def wait_scatter(b):
        pltpu.make_async_copy(outb[b], accs.at[dsts[b]], sems[b]).wait()

    def issue_scatter(b):
        for g in range(_CH // 16):
            dsts[b][pl.ds(g * 16, 16)] = dstv[b][pl.ds(g * 16, 16)]
        pltpu.async_copy(outb[b], accs.at[dsts[b]], sems[b], add=True)

    def compute_groups(q_r, kv_r, ep_r, o_r, ngroups):
        for g in range(ngroups):
            eidx = lanes + (g * 16)
            alpha = jnp.zeros((16,), _f32)
            ve = []
            for j in range(10):
                jj = jnp.full((16,), j, _i32)
                qj = plsc.load_gather(q_r, [eidx, jj])
                kj = plsc.load_gather(kv_r, [eidx, jj])
                ej = plsc.load_gather(ep_r, [eidx, jj])
                vj = plsc.load_gather(kv_r, [eidx, jj + 16])
                alpha = alpha + qj * (kj + ej)
                ve.append(vj + ej)
            ex = jnp.exp(alpha)
            for j in range(10):
                jj = jnp.full((16,), j, _i32)
                plsc.store_scatter(o_r, [eidx, jj], ex * ve[j])
            plsc.store_scatter(o_r, [eidx, jnp.full((16,), 10, _i32)], ex)

    issue_idx(0, 0)
    issue_idx(1, 1)
    issue_idx(2, 2)
    wait_idx(0)
    issue_gather(0)
    wait_idx(1)
    issue_gather(1)

    def _triple(j3, carry):
        for b in (0, 1, 2):
            j = j3 * 3 + b
            b2 = (b + 2) % 3

            @pl.when(j + 2 < _NFULL)
            def _():
                wait_idx(b2)
                issue_gather(b2)

            wait_gather(b)

            @pl.when(j >= 3)
            def _():
                wait_scatter(b)

            compute_groups(qv[b], kvv[b], epv[b], outb[b], 8)
            issue_scatter(b)

            @pl.when(j + 3 < _NFULL)
            def _():
                issue_idx(j + 3, b)
        return carry

    lax.fori_loop(0, _NFULL // 3, _triple, 0)
    for b in (0, 1, 2):
        wait_scatter(b)

    toff = base + _TAILOFF
    pltpu.sync_copy(src_hbm.at[pl.ds(toff, _TAIL)], srct)
    pltpu.sync_copy(dst_hbm.at[pl.ds(toff, _TAIL)], dstt)
    g1 = pltpu.async_copy(kv_s.at[srct], kvt, semt)
    pltpu.sync_copy(ep_hbm.at[pl.ds(toff, _TAIL)], ept)
    g1.wait()
    pltpu.async_copy(q_s.at[dstt], qt, semt).wait()
    def _zt(i, carry):
        outt[i, :] = jnp.zeros((16,), _f32)
        return carry
    lax.fori_loop(0, _TAIL, _zt, 0)
    compute_groups(qt, kvt, ept, outt, 1)
    pltpu.sync_copy(outt, accs.at[dstt], add=True)

    plsc.subcore_barrier()
    pltpu.sync_copy(accs.at[pl.ds(s * _RPS, _RPS)], zbuf)
    pltpu.sync_copy(zbuf, out_hbm.at[c, pl.ds(s * _RPS, _RPS)])


@functools.cache
def _edge_pass():
    return pl.kernel(
    _edge_pass_body,
    out_type=jax.ShapeDtypeStruct((2, _NPAD, 16), _f32),
    mesh=_sc_mesh(),
    scratch_types=(
        [pltpu.VMEM((_CH,), _i32)] * 9
        + [pltpu.VMEM((_CH, 32), _f32)] * 3
        + [pltpu.VMEM((_CH, 16), _f32)] * 3
        + [pltpu.VMEM((_CH, 16), _f32)] * 3
        + [pltpu.VMEM((_CH, 16), _f32)] * 3
        + [pltpu.VMEM((_TAIL,), _i32)] * 2
        + [pltpu.VMEM((_TAIL, 32), _f32)]
        + [pltpu.VMEM((_TAIL, 16), _f32)] * 3
        + [pltpu.VMEM((_RPS, 16), _f32)]
        + [pltpu.SemaphoreType.DMA] * 16
        + [pltpu.VMEM_SHARED((_NPAD, 32), _f32)]
        + [pltpu.VMEM_SHARED((_NPAD, 16), _f32)] * 2
    ),
    compiler_params=pltpu.CompilerParams(
        needs_layout_passes=False, use_tc_tiling_on_sc=False),
    )


def _pick_body(tab_hbm, idx_hbm, out_hbm, idxv, rowsv, sem):
    c = lax.axis_index("c")
    s = lax.axis_index("s")
    wid = s * 2 + c
    base = wid * 32
    pltpu.sync_copy(idx_hbm.at[pl.ds(base, 32)], idxv)
    pltpu.async_copy(tab_hbm.at[idxv], rowsv, sem).wait()
    pltpu.sync_copy(rowsv, out_hbm.at[pl.ds(base, 32)])


@functools.cache
def _pick():
    return pl.kernel(
        _pick_body,
        out_type=jax.ShapeDtypeStruct((1024, 16), _f32),
        mesh=_sc_mesh(),
        scratch_types=[
            pltpu.VMEM((32,), _i32),
            pltpu.VMEM((32, 16), _f32),
            pltpu.SemaphoreType.DMA,
        ],
        compiler_params=pltpu.CompilerParams(
            needs_layout_passes=False, use_tc_tiling_on_sc=False),
    )


def _pad_cols(w, cols):
    return jnp.concatenate([w, jnp.zeros((w.shape[0], cols - w.shape[1]), _f32)], axis=1)


def _qkvs_weights(tc):
    rs = np.float32(1.0 / np.sqrt(10.0))
    z6 = jnp.zeros((138, 6), _f32)
    w = jnp.concatenate(
        [tc["Wq"] * rs, z6, tc["Wk"], z6, tc["Wv"], z6, tc["Ws"], z6], axis=1)
    w = jnp.concatenate([w, jnp.zeros((6, 64), _f32)], axis=0)
    z6b = jnp.zeros((6,), _f32)
    b = jnp.concatenate(
        [tc["bq"] * rs, z6b, tc["bk"], z6b, tc["bv"], z6b, tc["bs"], z6b])
    return w, b.reshape(1, 64)


def kernel(x, z, edge_index, z1edge_index, z2edge_index, z3edge_index,
           z4edge_index, z5edge_index, edge_attr, pickable, params):
    p = params
    f32 = _f32

    l1w = _pad_cols(p["l1_W"], 16)
    l1b = _pad_cols(p["l1_b"].reshape(1, 10), 16)
    eaw = _pad_cols(p["eA_W"], 16)
    eab = _pad_cols(p["eA_b"].reshape(1, 10), 16)
    e1w = _pad_cols(p["e1_W"], 16)
    e1b = _pad_cols(p["e1_b"].reshape(1, 10), 16)
    e2w = jnp.zeros((16, 16), f32).at[:10, :10].set(p["e2_W"])
    e2b = _pad_cols(p["e2_b"].reshape(1, 10), 16)
    wes = {l: jnp.zeros((16, 16), f32).at[:10, :10].set(p[f"tc{l}"]["We"])
           for l in (2, 3, 4, 5)}
    qkvs = {l: _qkvs_weights(p[f"tc{l}"]) for l in (2, 3, 4, 5)}
    linw = jnp.zeros((16, 16), f32).at[:10, :10].set(p["lin_W"])
    linb = jnp.full((1, 16), -1e30, f32).at[0, :10].set(p["lin_b"])

    xn, h144 = pl.pallas_call(
        _node_prelude_body,
        out_shape=[jax.ShapeDtypeStruct((_NPAD, 128), f32),
                   jax.ShapeDtypeStruct((_NPAD, 144), f32)],
    )(x, p["l0_W"], p["l0_b"].reshape(1, 128), p["n_w"].reshape(1, 128),
      p["n_b"].reshape(1, 128), p["n_ms"].reshape(1, 128), l1w, l1b)

    ngrid = _E // _EBLK
    blk = pl.BlockSpec((_EBLK, 16), lambda i: (i, 0))
    wspec = pl.BlockSpec((16, 16), lambda i: (0, 0))
    bspec = pl.BlockSpec((1, 16), lambda i: (0, 0))
    ea, stats = pl.pallas_call(
        _edge_stats_body,
        grid=(ngrid,),
        in_specs=[blk, wspec, bspec],
        out_specs=[blk, pl.BlockSpec((8, 16), lambda i: (0, 0))],
        out_shape=[jax.ShapeDtypeStruct((_E, 16), f32),
                   jax.ShapeDtypeStruct((8, 16), f32)],
    )(edge_attr, _pad_cols(p["l0e_W"], 16)[:16, :],
      _pad_cols(p["l0e_b"].reshape(1, 16), 16))

    eps = pl.pallas_call(
        _edge_proj_body,
        grid=(ngrid,),
        in_specs=[blk, pl.BlockSpec((8, 16), lambda i: (0, 0)),
                  bspec, bspec, bspec,
                  wspec, bspec, wspec, bspec, wspec, bspec,
                  wspec, wspec, wspec, wspec],
        out_specs=[blk, blk, blk, blk],
        out_shape=[jax.ShapeDtypeStruct((_E, 16), f32)] * 4,
    )(ea, stats,
      _pad_cols(p["n2_w"].reshape(1, 16), 16),
      _pad_cols(p["n2_b"].reshape(1, 16), 16),
      _pad_cols(p["n2_ms"].reshape(1, 16), 16),
      eaw, eab, e1w, e1b, e2w, e2b,
      wes[2], wes[3], wes[4], wes[5])

    src = edge_index[0]
    dst = edge_index[1]

    nblk = 2048
    rgrid = (_NPAD // nblk,)
    b144 = pl.BlockSpec((nblk, 144), lambda i: (i, 0))
    b128 = pl.BlockSpec((nblk, 128), lambda i: (i, 0))
    b64w = pl.BlockSpec((144, 64), lambda i: (0, 0))
    b64b = pl.BlockSpec((1, 64), lambda i: (0, 0))
    b16 = pl.BlockSpec((nblk, 16), lambda i: (i, 0))
    b32 = pl.BlockSpec((nblk, 32), lambda i: (i, 0))
    bacc = pl.BlockSpec((2, nblk, 16), lambda i: (0, i, 0))
    w16 = pl.BlockSpec((16, 16), lambda i: (0, 0))
    w1x16 = pl.BlockSpec((1, 16), lambda i: (0, 0))
    proj_out_shape = [jax.ShapeDtypeStruct((_NPAD, 16), f32),
                      jax.ShapeDtypeStruct((_NPAD, 32), f32),
                      jax.ShapeDtypeStruct((_NPAD, 16), f32)]
    proj_out_specs = [b16, b32, b16]

    q_t, kv_t, s_t = pl.pallas_call(
        _proj_body,
        grid=rgrid,
        in_specs=[b144, b64w, b64b],
        out_specs=proj_out_specs,
        out_shape=proj_out_shape,
    )(h144, qkvs[2][0], qkvs[2][1])

    for l in (2, 3, 4):
        acc = _edge_pass()(src, dst, kv_t, q_t, eps[l - 2])
        q_t, kv_t, s_t = pl.pallas_call(
            _combine_proj_body,
            grid=rgrid,
            in_specs=[bacc, b16, b128, b64w, b64b],
            out_specs=proj_out_specs,
            out_shape=proj_out_shape,
        )(acc, s_t, xn, qkvs[l + 1][0], qkvs[l + 1][1])

    acc = _edge_pass()(src, dst, kv_t, q_t, eps[3])
    logits = pl.pallas_call(
        _combine_final_body,
        grid=rgrid,
        in_specs=[bacc, b16, w16, w1x16],
        out_specs=b16,
        out_shape=jax.ShapeDtypeStruct((_NPAD, 16), f32),
    )(acc, s_t, linw, linb)

    pick_pad = jnp.concatenate([pickable, jnp.zeros((24,), _i32)])
    picked = _pick()(logits, pick_pad)
    sm = pl.pallas_call(
        _softmax_body,
        out_shape=jax.ShapeDtypeStruct((1024, 16), f32),
    )(picked)
    return sm[:1000, :10]


# speedup vs baseline: 1.0259x; 1.0044x over previous
import functools
import numpy as np
import jax
import jax.numpy as jnp
from jax import lax
from jax.experimental import pallas as pl
from jax.experimental.pallas import tpu as pltpu
from jax.experimental.pallas import tpu_sc as plsc

_N = 10000
_E = 320000
_NPAD = 10240
_NW = 32
_EPW = _E // _NW
_CH = 128
_NFULL = _EPW // _CH
_TAILOFF = _NFULL * _CH
_TAIL = _EPW - _TAILOFF
_RPS = _NPAD // 16
_EBLK = 2000
_HI = lax.Precision.HIGHEST

_f32 = jnp.float32
_i32 = jnp.int32


def _node_prelude_body(x_ref, w0_ref, b0_ref, nw_ref, nb_ref, nms_ref,
                       w1_ref, b1_ref, xn_ref, h_ref):
    x1 = jnp.dot(x_ref[...], w0_ref[...], precision=_HI) + b0_ref[...]
    mean = jnp.mean(x1, axis=0, keepdims=True)
    o = x1 - mean * nms_ref[...]
    var = jnp.mean(o * o, axis=0, keepdims=True)
    xn = nw_ref[...] * o / jnp.sqrt(var + 1e-5) + nb_ref[...]
    h = jnp.maximum(jnp.dot(xn, w1_ref[...], precision=_HI) + b1_ref[...], 0.0)
    zrows = jnp.zeros((_NPAD - _N, 128), _f32)
    zrows16 = jnp.zeros((_NPAD - _N, 16), _f32)
    xn_ref[...] = jnp.concatenate([xn, zrows], axis=0)
    h_ref[...] = jnp.concatenate(
        [jnp.concatenate([xn, h], axis=1),
         jnp.concatenate([zrows, zrows16], axis=1)], axis=0)


def _edge_stats_body(ea_ref, w_ref, b_ref, out_ref, st_ref):
    i = pl.program_id(0)
    ea = jnp.dot(ea_ref[...], w_ref[...], precision=_HI) + b_ref[...]
    out_ref[...] = ea

    @pl.when(i == 0)
    def _():
        st_ref[...] = jnp.zeros_like(st_ref)

    s = jnp.sum(ea, axis=0, keepdims=True)
    s2 = jnp.sum(ea * ea, axis=0, keepdims=True)
    st_ref[...] += jnp.concatenate(
        [s, s2, jnp.zeros((6, 16), _f32)], axis=0)


def _edge_proj_body(ea_ref, st_ref, n2w_ref, n2b_ref, n2ms_ref,
                    eaw_ref, eab_ref, e1w_ref, e1b_ref, e2w_ref, e2b_ref,
                    we2_ref, we3_ref, we4_ref, we5_ref,
                    o2_ref, o3_ref, o4_ref, o5_ref):
    st = st_ref[...]
    m = st[0:1, :] / _E
    msq = st[1:2, :] / _E
    ms = n2ms_ref[...]
    var = msq - 2.0 * ms * m * m + ms * ms * m * m
    o = ea_ref[...] - m * ms
    ean = n2w_ref[...] * o / jnp.sqrt(var + 1e-5) + n2b_ref[...]
    ea_b = jnp.dot(ean, eaw_ref[...], precision=_HI) + eab_ref[...]
    h10 = jnp.maximum(jnp.dot(ean, e1w_ref[...], precision=_HI) + e1b_ref[...], 0.0)
    ea10 = jnp.dot(h10, e2w_ref[...], precision=_HI) + e2b_ref[...]
    col = lax.broadcasted_iota(_i32, (_EBLK, 16), 1)
    for dst_ref, src, w_ref in ((o2_ref, ea10, we2_ref), (o3_ref, ea10, we3_ref),
                                (o4_ref, ea_b, we4_ref), (o5_ref, ea_b, we5_ref)):
        ep = jnp.dot(src, w_ref[...], precision=_HI)
        dst_ref[...] = jnp.where(col == 10, 1.0, ep)


def _proj_core(h, w_ref, b_ref, q_ref, kv_ref, s_ref):
    o = jnp.dot(h, w_ref[...], precision=_HI) + b_ref[...]
    q_ref[...] = o[:, 0:16]
    kv_ref[...] = o[:, 16:48]
    s_ref[...] = o[:, 48:64]


def _proj_body(h_ref, w_ref, b_ref, q_ref, kv_ref, s_ref):
    _proj_core(h_ref[...], w_ref, b_ref, q_ref, kv_ref, s_ref)


def _combine16(acc_ref, s_ref):
    num = acc_ref[0] + acc_ref[1]
    den = num[:, 10:11]
    return num / (den + 1e-16) + s_ref[...]


def _combine_proj_body(acc_ref, s_ref, xn_ref, w_ref, b_ref,
                       q_ref, kv_ref, s2_ref):
    o16 = jnp.maximum(_combine16(acc_ref, s_ref), 0.0)
    h = jnp.concatenate([xn_ref[...], o16], axis=1)
    _proj_core(h, w_ref, b_ref, q_ref, kv_ref, s2_ref)


def _combine_final_body(acc_ref, s_ref, lw_ref, lb_ref, out_ref):
    o16 = _combine16(acc_ref, s_ref)
    out_ref[...] = jnp.dot(o16, lw_ref[...], precision=_HI) + lb_ref[...]


def _softmax_body(g_ref, out_ref):
    g = g_ref[...]
    m = jnp.max(g, axis=1, keepdims=True)
    e = jnp.exp(g - m)
    out_ref[...] = e / jnp.sum(e, axis=1, keepdims=True)


@functools.cache
def _sc_mesh():
    return plsc.VectorSubcoreMesh(core_axis_name="c", subcore_axis_name="s")


def _edge_pass_body(src_hbm, dst_hbm, kv_hbm, q_hbm, ep_hbm, out_hbm,
                    srcv0, srcv1, srcv2, dstv0, dstv1, dstv2,
                    dsts0, dsts1, dsts2,
                    kvv0, kvv1, kvv2, qv0, qv1, qv2,
                    epv0, epv1, epv2, outb0, outb1, outb2,
                    srct, dstt, kvt, qt, ept, outt,
                    zbuf, semi0, semi1, semi2, semk0, semk1, semk2,
                    semq0, semq1, semq2, seme0, seme1, seme2,
                    sems0, sems1, sems2, semt, kv_s, q_s, accs):
    c = lax.axis_index("c")
    s = lax.axis_index("s")
    wid = s * 2 + c
    base = wid * _EPW
    lanes = lax.iota(_i32, 16)
    srcv = (srcv0, srcv1, srcv2)
    dstv = (dstv0, dstv1, dstv2)
    dsts = (dsts0, dsts1, dsts2)
    kvv = (kvv0, kvv1, kvv2)
    qv = (qv0, qv1, qv2)
    epv = (epv0, epv1, epv2)
    outb = (outb0, outb1, outb2)
    semi = (semi0, semi1, semi2)
    semk = (semk0, semk1, semk2)
    semq = (semq0, semq1, semq2)
    seme = (seme0, seme1, seme2)
    sems = (sems0, sems1, sems2)

    def _zrow(i, carry):
        zbuf[i, :] = jnp.zeros((16,), _f32)
        return carry

    lax.fori_loop(0, _RPS, _zrow, 0)
    pltpu.sync_copy(zbuf, accs.at[pl.ds(s * _RPS, _RPS)])
    pltpu.sync_copy(kv_hbm.at[pl.ds(s * _RPS, _RPS)], kv_s.at[pl.ds(s * _RPS, _RPS)])
    pltpu.sync_copy(q_hbm.at[pl.ds(s * _RPS, _RPS)], q_s.at[pl.ds(s * _RPS, _RPS)])
    for b in (0, 1, 2):
        def _z16(i, carry, _b=b):
            outb[_b][i, :] = jnp.zeros((16,), _f32)
            return carry
        lax.fori_loop(0, _CH, _z16, 0)
    plsc.subcore_barrier()

    def issue_idx(j, b):
        off = base + j * _CH
        pltpu.async_copy(src_hbm.at[pl.ds(off, _CH)], srcv[b], semi[b])
        pltpu.async_copy(dst_hbm.at[pl.ds(off, _CH)], dstv[b], semi[b])
        pltpu.async_copy(ep_hbm.at[pl.ds(off, _CH)], epv[b], seme[b])

    def wait_idx(b):
        pltpu.make_async_copy(src_hbm.at[pl.ds(base, _CH)], srcv[b], semi[b]).wait()
        pltpu.make_async_copy(dst_hbm.at[pl.ds(base, _CH)], dstv[b], semi[b]).wait()

    def issue_gather(b):
        pltpu.async_copy(kv_s.at[srcv[b]], kvv[b], semk[b])
        pltpu.async_copy(q_s.at[dstv[b]], qv[b], semq[b])

    def wait_gather(b):
        pltpu.make_async_copy(kv_s.at[srcv[b]], kvv[b], semk[b]).wait()
        pltpu.make_async_copy(q_s.at[dstv[b]], qv[b], semq[b]).wait()
        pltpu.make_async_copy(ep_hbm.at[pl.ds(base, _CH)], epv[b], seme[b]).wait()

    def wait_scatter(b):
        pltpu.make_async_copy(outb[b], accs.at[dsts[b]], sems[b]).wait()

    def issue_scatter(b):
        for g in range(_CH // 16):
            dsts[b][pl.ds(g * 16, 16)] = dstv[b][pl.ds(g * 16, 16)]
        pltpu.async_copy(outb[b], accs.at[dsts[b]], sems[b], add=True)

    def compute_groups(q_r, kv_r, ep_r, o_r, ngroups):
        for g in range(ngroups):
            rows = lanes + (g * 16)
            alpha = jnp.zeros((16,), _f32)
            ve = []
            for d in range(16):
                cols = jnp.bitwise_and(lanes + d, 15)
                qd = plsc.load_gather(q_r, [rows, cols])
                kd = plsc.load_gather(kv_r, [rows, cols])
                ed = plsc.load_gather(ep_r, [rows, cols])
                vd = plsc.load_gather(kv_r, [rows, cols + 16])
                alpha = alpha + qd * (kd + ed)
                ve.append(vd + ed)
            ex = jnp.exp(alpha)
            for d in range(16):
                cols = jnp.bitwise_and(lanes + d, 15)
                plsc.store_scatter(o_r, [rows, cols], ex * ve[d])

    issue_idx(0, 0)
    issue_idx(1, 1)
    issue_idx(2, 2)
    wait_idx(0)
    issue_gather(0)
    wait_idx(1)
    issue_gather(1)

    def _triple(j3, carry):
        for b in (0, 1, 2):
            j = j3 * 3 + b
            b2 = (b + 2) % 3

            @pl.when(j + 2 < _NFULL)
            def _():
                wait_idx(b2)
                issue_gather(b2)

            wait_gather(b)

            @pl.when(j >= 3)
            def _():
                wait_scatter(b)

            compute_groups(qv[b], kvv[b], epv[b], outb[b], 8)
            issue_scatter(b)

            @pl.when(j + 3 < _NFULL)
            def _():
                issue_idx(j + 3, b)
        return carry

    lax.fori_loop(0, _NFULL // 3, _triple, 0)
    for b in (0, 1, 2):
        wait_scatter(b)

    toff = base + _TAILOFF
    pltpu.sync_copy(src_hbm.at[pl.ds(toff, _TAIL)], srct)
    pltpu.sync_copy(dst_hbm.at[pl.ds(toff, _TAIL)], dstt)
    g1 = pltpu.async_copy(kv_s.at[srct], kvt, semt)
    pltpu.sync_copy(ep_hbm.at[pl.ds(toff, _TAIL)], ept)
    g1.wait()
    pltpu.async_copy(q_s.at[dstt], qt, semt).wait()
    def _zt(i, carry):
        outt[i, :] = jnp.zeros((16,), _f32)
        return carry
    lax.fori_loop(0, _TAIL, _zt, 0)
    compute_groups(qt, kvt, ept, outt, 1)
    pltpu.sync_copy(outt, accs.at[dstt], add=True)

    plsc.subcore_barrier()
    pltpu.sync_copy(accs.at[pl.ds(s * _RPS, _RPS)], zbuf)
    pltpu.sync_copy(zbuf, out_hbm.at[c, pl.ds(s * _RPS, _RPS)])


@functools.cache
def _edge_pass():
    return pl.kernel(
    _edge_pass_body,
    out_type=jax.ShapeDtypeStruct((2, _NPAD, 16), _f32),
    mesh=_sc_mesh(),
    scratch_types=(
        [pltpu.VMEM((_CH,), _i32)] * 9
        + [pltpu.VMEM((_CH, 32), _f32)] * 3
        + [pltpu.VMEM((_CH, 16), _f32)] * 3
        + [pltpu.VMEM((_CH, 16), _f32)] * 3
        + [pltpu.VMEM((_CH, 16), _f32)] * 3
        + [pltpu.VMEM((_TAIL,), _i32)] * 2
        + [pltpu.VMEM((_TAIL, 32), _f32)]
        + [pltpu.VMEM((_TAIL, 16), _f32)] * 3
        + [pltpu.VMEM((_RPS, 16), _f32)]
        + [pltpu.SemaphoreType.DMA] * 16
        + [pltpu.VMEM_SHARED((_NPAD, 32), _f32)]
        + [pltpu.VMEM_SHARED((_NPAD, 16), _f32)] * 2
    ),
    compiler_params=pltpu.CompilerParams(
        needs_layout_passes=False, use_tc_tiling_on_sc=False),
    )


def _pick_body(tab_hbm, idx_hbm, out_hbm, idxv, rowsv, sem):
    c = lax.axis_index("c")
    s = lax.axis_index("s")
    wid = s * 2 + c
    base = wid * 32
    pltpu.sync_copy(idx_hbm.at[pl.ds(base, 32)], idxv)
    pltpu.async_copy(tab_hbm.at[idxv], rowsv, sem).wait()
    pltpu.sync_copy(rowsv, out_hbm.at[pl.ds(base, 32)])


@functools.cache
def _pick():
    return pl.kernel(
        _pick_body,
        out_type=jax.ShapeDtypeStruct((1024, 16), _f32),
        mesh=_sc_mesh(),
        scratch_types=[
            pltpu.VMEM((32,), _i32),
            pltpu.VMEM((32, 16), _f32),
            pltpu.SemaphoreType.DMA,
        ],
        compiler_params=pltpu.CompilerParams(
            needs_layout_passes=False, use_tc_tiling_on_sc=False),
    )


def _pad_cols(w, cols):
    return jnp.concatenate([w, jnp.zeros((w.shape[0], cols - w.shape[1]), _f32)], axis=1)


def _qkvs_weights(tc):
    rs = np.float32(1.0 / np.sqrt(10.0))
    z6 = jnp.zeros((138, 6), _f32)
    w = jnp.concatenate(
        [tc["Wq"] * rs, z6, tc["Wk"], z6, tc["Wv"], z6, tc["Ws"], z6], axis=1)
    w = jnp.concatenate([w, jnp.zeros((6, 64), _f32)], axis=0)
    z6b = jnp.zeros((6,), _f32)
    b = jnp.concatenate(
        [tc["bq"] * rs, z6b, tc["bk"], z6b, tc["bv"], z6b, tc["bs"], z6b])
    return w, b.reshape(1, 64)


def kernel(x, z, edge_index, z1edge_index, z2edge_index, z3edge_index,
           z4edge_index, z5edge_index, edge_attr, pickable, params):
    p = params
    f32 = _f32

    l1w = _pad_cols(p["l1_W"], 16)
    l1b = _pad_cols(p["l1_b"].reshape(1, 10), 16)
    eaw = _pad_cols(p["eA_W"], 16)
    eab = _pad_cols(p["eA_b"].reshape(1, 10), 16)
    e1w = _pad_cols(p["e1_W"], 16)
    e1b = _pad_cols(p["e1_b"].reshape(1, 10), 16)
    e2w = jnp.zeros((16, 16), f32).at[:10, :10].set(p["e2_W"])
    e2b = _pad_cols(p["e2_b"].reshape(1, 10), 16)
    wes = {l: jnp.zeros((16, 16), f32).at[:10, :10].set(p[f"tc{l}"]["We"])
           for l in (2, 3, 4, 5)}
    qkvs = {l: _qkvs_weights(p[f"tc{l}"]) for l in (2, 3, 4, 5)}
    linw = jnp.zeros((16, 16), f32).at[:10, :10].set(p["lin_W"])
    linb = jnp.full((1, 16), -1e30, f32).at[0, :10].set(p["lin_b"])

    xn, h144 = pl.pallas_call(
        _node_prelude_body,
        out_shape=[jax.ShapeDtypeStruct((_NPAD, 128), f32),
                   jax.ShapeDtypeStruct((_NPAD, 144), f32)],
    )(x, p["l0_W"], p["l0_b"].reshape(1, 128), p["n_w"].reshape(1, 128),
      p["n_b"].reshape(1, 128), p["n_ms"].reshape(1, 128), l1w, l1b)

    ngrid = _E // _EBLK
    blk = pl.BlockSpec((_EBLK, 16), lambda i: (i, 0))
    wspec = pl.BlockSpec((16, 16), lambda i: (0, 0))
    bspec = pl.BlockSpec((1, 16), lambda i: (0, 0))
    ea, stats = pl.pallas_call(
        _edge_stats_body,
        grid=(ngrid,),
        in_specs=[blk, wspec, bspec],
        out_specs=[blk, pl.BlockSpec((8, 16), lambda i: (0, 0))],
        out_shape=[jax.ShapeDtypeStruct((_E, 16), f32),
                   jax.ShapeDtypeStruct((8, 16), f32)],
    )(edge_attr, _pad_cols(p["l0e_W"], 16)[:16, :],
      _pad_cols(p["l0e_b"].reshape(1, 16), 16))

    eps = pl.pallas_call(
        _edge_proj_body,
        grid=(ngrid,),
        in_specs=[blk, pl.BlockSpec((8, 16), lambda i: (0, 0)),
                  bspec, bspec, bspec,
                  wspec, bspec, wspec, bspec, wspec, bspec,
                  wspec, wspec, wspec, wspec],
        out_specs=[blk, blk, blk, blk],
        out_shape=[jax.ShapeDtypeStruct((_E, 16), f32)] * 4,
    )(ea, stats,
      _pad_cols(p["n2_w"].reshape(1, 16), 16),
      _pad_cols(p["n2_b"].reshape(1, 16), 16),
      _pad_cols(p["n2_ms"].reshape(1, 16), 16),
      eaw, eab, e1w, e1b, e2w, e2b,
      wes[2], wes[3], wes[4], wes[5])

    src = edge_index[0]
    dst = edge_index[1]

    nblk = 2048
    rgrid = (_NPAD // nblk,)
    b144 = pl.BlockSpec((nblk, 144), lambda i: (i, 0))
    b128 = pl.BlockSpec((nblk, 128), lambda i: (i, 0))
    b64w = pl.BlockSpec((144, 64), lambda i: (0, 0))
    b64b = pl.BlockSpec((1, 64), lambda i: (0, 0))
    b16 = pl.BlockSpec((nblk, 16), lambda i: (i, 0))
    b32 = pl.BlockSpec((nblk, 32), lambda i: (i, 0))
    bacc = pl.BlockSpec((2, nblk, 16), lambda i: (0, i, 0))
    w16 = pl.BlockSpec((16, 16), lambda i: (0, 0))
    w1x16 = pl.BlockSpec((1, 16), lambda i: (0, 0))
    proj_out_shape = [jax.ShapeDtypeStruct((_NPAD, 16), f32),
                      jax.ShapeDtypeStruct((_NPAD, 32), f32),
                      jax.ShapeDtypeStruct((_NPAD, 16), f32)]
    proj_out_specs = [b16, b32, b16]

    q_t, kv_t, s_t = pl.pallas_call(
        _proj_body,
        grid=rgrid,
        in_specs=[b144, b64w, b64b],
        out_specs=proj_out_specs,
        out_shape=proj_out_shape,
    )(h144, qkvs[2][0], qkvs[2][1])

    for l in (2, 3, 4):
        acc = _edge_pass()(src, dst, kv_t, q_t, eps[l - 2])
        q_t, kv_t, s_t = pl.pallas_call(
            _combine_proj_body,
            grid=rgrid,
            in_specs=[bacc, b16, b128, b64w, b64b],
            out_specs=proj_out_specs,
            out_shape=proj_out_shape,
        )(acc, s_t, xn, qkvs[l + 1][0], qkvs[l + 1][1])

    acc = _edge_pass()(src, dst, kv_t, q_t, eps[3])
    logits = pl.pallas_call(
        _combine_final_body,
        grid=rgrid,
        in_specs=[bacc, b16, w16, w1x16],
        out_specs=b16,
        out_shape=jax.ShapeDtypeStruct((_NPAD, 16), f32),
    )(acc, s_t, linw, linb)

    pick_pad = jnp.concatenate([pickable, jnp.zeros((24,), _i32)])
    picked = _pick()(logits, pick_pad)
    sm = pl.pallas_call(
        _softmax_body,
        out_shape=jax.ShapeDtypeStruct((1024, 16), f32),
    )(picked)
    return sm[:1000, :10]
